# baseline (speedup 1.0000x reference)
import functools

import jax
import jax.numpy as jnp
from jax import lax
from jax.experimental import pallas as pl
from jax.experimental.pallas import tpu as pltpu
from jax.experimental.pallas import tpu_sc as plsc

NUM_QUBITS = 4096
NUM_CORES = 16
CORE_SIZE = 512
HIDDEN = 128
B = 64
MINF = -3.0e38

CH = 256
NCH = NUM_QUBITS // CH
CHW = CH * HIDDEN
NVR = HIDDEN // 16
BPW = 2


def _sc_body(a_hbm, emb_hbm, s_hbm, a_v, eb0, eb1, acc_v, s_v, cnt_s,
             sem0, sem1, sem_a):
    nc = 2
    wid = lax.axis_index("s") * nc + lax.axis_index("c")
    b0 = wid * BPW

    cp_a = pltpu.make_async_copy(
        a_hbm.at[pl.ds(b0 * NUM_QUBITS, BPW * NUM_QUBITS)], a_v, sem_a)
    cp_a.start()

    ebs = [eb0, eb1]
    sems = [sem0, sem1]

    def chunk_copy(k, buf):
        return pltpu.make_async_copy(
            emb_hbm.at[pl.ds(k * CHW, CHW)], ebs[buf].at[pl.ds(0, CHW)],
            sems[buf])

    chunk_copy(0, 0).start()

    minf16 = jnp.full((16,), MINF, jnp.float32)
    for j in range(NVR):
        eb0[pl.ds(CHW + j * 16, 16)] = minf16
        eb1[pl.ds(CHW + j * 16, 16)] = minf16

    def acc_init(i, _):
        acc_v[pl.ds(i * 16, 16)] = minf16
        return 0
    lax.fori_loop(0, BPW * NUM_CORES * HIDDEN // 16, acc_init, 0)

    def cnt_init(i, _):
        cnt_s[i] = 0
        return 0
    lax.fori_loop(0, BPW * NUM_CORES, cnt_init, 0)

    cp_a.wait()

    def process_chunk(k, eb):
        for i in range(BPW):
            abase = i * NUM_QUBITS + k * CH
            cbase = i * NUM_CORES
            dbase = i * NUM_CORES * HIDDEN

            def gbody(g, _, abase=abase, cbase=cbase, dbase=dbase, eb=eb):
                q0 = g * 16
                cvec = a_v[pl.ds(abase + q0, 16)]
                for l in range(16):
                    c = cvec[l]
                    cc = cbase + c
                    cnt = cnt_s[cc]
                    cnt_s[cc] = cnt + 1
                    src = jnp.where(cnt < CORE_SIZE, (q0 + l) * HIDDEN, CHW)
                    dst = dbase + c * HIDDEN
                    for j in range(NVR):
                        v = eb[pl.ds(src + j * 16, 16)]
                        w = acc_v[pl.ds(dst + j * 16, 16)]
                        acc_v[pl.ds(dst + j * 16, 16)] = jnp.maximum(w, v)
                return 0

            lax.fori_loop(0, CH // 16, gbody, 0)

    def pair_body(k2, _):
        k0 = 2 * k2
        chunk_copy(k0, 0).wait()
        chunk_copy(k0 + 1, 1).start()
        process_chunk(k0, eb0)
        chunk_copy(k0 + 1, 1).wait()

        @pl.when(k0 + 2 < NCH)
        def _():
            chunk_copy(k0 + 2, 0).start()

        process_chunk(k0 + 1, eb1)
        return 0

    lax.fori_loop(0, NCH // 2, pair_body, 0)

    zero16 = jnp.zeros((16,), jnp.float32)
    for i in range(BPW):
        sj = [zero16 for _ in range(NVR)]
        for c in range(NUM_CORES):
            full_flag = cnt_s[i * NUM_CORES + c] >= CORE_SIZE
            for j in range(NVR):
                row = acc_v[pl.ds(i * NUM_CORES * HIDDEN + c * HIDDEN + j * 16, 16)]
                row = jnp.where(full_flag, row, jnp.maximum(row, 0.0))
                sj[j] = sj[j] + row
        for j in range(NVR):
            s_v[pl.ds(i * HIDDEN + j * 16, 16)] = sj[j]
    pltpu.sync_copy(s_v, s_hbm.at[pl.ds(b0 * HIDDEN, BPW * HIDDEN)])


def _segmax_sums(last_assignment, emb_table):
    mesh = plsc.VectorSubcoreMesh(core_axis_name="c", subcore_axis_name="s")
    fn = functools.partial(
        pl.kernel,
        mesh=mesh,
        out_type=jax.ShapeDtypeStruct((B * HIDDEN,), jnp.float32),
        scratch_types=[
            pltpu.VMEM((BPW * NUM_QUBITS,), jnp.int32),
            pltpu.VMEM((CHW + HIDDEN,), jnp.float32),
            pltpu.VMEM((CHW + HIDDEN,), jnp.float32),
            pltpu.VMEM((BPW * NUM_CORES * HIDDEN,), jnp.float32),
            pltpu.VMEM((BPW * HIDDEN,), jnp.float32),
            pltpu.SMEM((BPW * NUM_CORES,), jnp.int32),
            pltpu.SemaphoreType.DMA,
            pltpu.SemaphoreType.DMA,
            pltpu.SemaphoreType.DMA,
        ],
    )(_sc_body)
    a_flat = last_assignment.reshape(B * NUM_QUBITS)
    emb_flat = emb_table[:NUM_QUBITS].reshape(NUM_QUBITS * HIDDEN)
    return fn(a_flat, emb_flat).reshape(B, HIDDEN)


def _mm_body(s_ref, W_ref, b_ref, out_ref):
    y = jnp.dot(s_ref[...], W_ref[...], preferred_element_type=jnp.float32)
    y = y * (1.0 / NUM_CORES) + b_ref[...]
    out_ref[...] = jnp.broadcast_to(y[:, None, :], (B, NUM_CORES, HIDDEN))


def kernel(last_assignment, emb_table, W, b):
    s = _segmax_sums(last_assignment, emb_table)
    out = pl.pallas_call(
        _mm_body,
        out_shape=jax.ShapeDtypeStruct((B, NUM_CORES, HIDDEN), jnp.float32),
    )(s, W, b.reshape(1, HIDDEN))
    return out
